# Initial kernel scaffold; baseline (speedup 1.0000x reference)
import jax
import jax.numpy as jnp
from jax.experimental import pallas as pl


def kernel(node_features, memory, source_nodes, timestamps, time_w, time_b):
    raise NotImplementedError("write your pallas kernel here")


# speedup vs baseline: 6.9051x; 6.9051x over previous
import functools

import jax
import jax.numpy as jnp
from jax import lax
from jax.experimental import pallas as pl
from jax.experimental.pallas import tpu as pltpu
from jax.experimental.pallas import tpu_sc as plsc

V = 100000
D = 128
B = 500000

NC, NS = 2, 16
NW = NC * NS
C = 400
SUB = 100
KSUB = C // SUB
NCHUNK = B // C

_ADD_BLOCK = 2000


def _add_body(a_ref, b_ref, o_ref):
    o_ref[...] = a_ref[...] + b_ref[...]


def _fuse_tables(nf, mem):
    return pl.pallas_call(
        _add_body,
        grid=(V // _ADD_BLOCK,),
        in_specs=[pl.BlockSpec((_ADD_BLOCK, D), lambda i: (i, 0))] * 2,
        out_specs=pl.BlockSpec((_ADD_BLOCK, D), lambda i: (i, 0)),
        out_shape=jax.ShapeDtypeStruct((V, D), jnp.float32),
    )(nf, mem)


_MESH = plsc.VectorSubcoreMesh(
    core_axis_name="c", subcore_axis_name="s", num_cores=NC, num_subcores=NS
)


@functools.partial(
    pl.kernel,
    out_type=jax.ShapeDtypeStruct((B, D), jnp.float32),
    mesh=_MESH,
    scratch_types=[
        pltpu.VMEM((KSUB, SUB), jnp.int32),
        pltpu.VMEM((C, D), jnp.float32),
        pltpu.SemaphoreType.DMA,
    ],
)
def _gather_k(table_hbm, idx_hbm, out_hbm, idx_v, rows_v, sem):
    wid = lax.axis_index("s") * NC + lax.axis_index("c")
    n_mine = (NCHUNK - wid + NW - 1) // NW

    def body(i, carry):
        j = wid + i * NW
        pltpu.sync_copy(idx_hbm.at[pl.ds(j * KSUB, KSUB), :], idx_v)
        copies = [
            pltpu.async_copy(
                table_hbm.at[idx_v.at[k]],
                rows_v.at[pl.ds(k * SUB, SUB), :],
                sem,
            )
            for k in range(KSUB)
        ]
        for cp in copies:
            cp.wait()
        pltpu.sync_copy(rows_v, out_hbm.at[pl.ds(j * C, C), :])
        return carry

    lax.fori_loop(0, n_mine, body, 0)


def kernel(node_features, memory, source_nodes, timestamps, time_w, time_b):
    del timestamps, time_w, time_b
    fused = _fuse_tables(node_features, memory)
    idx = source_nodes.astype(jnp.int32).reshape(NCHUNK * KSUB, SUB)
    return _gather_k(fused, idx)


# speedup vs baseline: 7.9216x; 1.1472x over previous
import functools

import jax
import jax.numpy as jnp
from jax import lax
from jax.experimental import pallas as pl
from jax.experimental.pallas import tpu as pltpu
from jax.experimental.pallas import tpu_sc as plsc

V = 100000
D = 128
B = 500000

NC, NS = 2, 16
NW = NC * NS
C = 400
SUB = 100
KSUB = C // SUB
NCHUNK = B // C

_ADD_BLOCK = 2000


def _add_body(a_ref, b_ref, o_ref):
    o_ref[...] = a_ref[...] + b_ref[...]


def _fuse_tables(nf, mem):
    return pl.pallas_call(
        _add_body,
        grid=(V // _ADD_BLOCK,),
        in_specs=[pl.BlockSpec((_ADD_BLOCK, D), lambda i: (i, 0))] * 2,
        out_specs=pl.BlockSpec((_ADD_BLOCK, D), lambda i: (i, 0)),
        out_shape=jax.ShapeDtypeStruct((V, D), jnp.float32),
    )(nf, mem)


_MESH = plsc.VectorSubcoreMesh(
    core_axis_name="c", subcore_axis_name="s", num_cores=NC, num_subcores=NS
)


@functools.partial(
    pl.kernel,
    out_type=jax.ShapeDtypeStruct((B, D), jnp.float32),
    mesh=_MESH,
    scratch_types=[
        pltpu.VMEM((KSUB, SUB), jnp.int32),
        pltpu.VMEM((KSUB, SUB), jnp.int32),
        pltpu.VMEM((C, D), jnp.float32),
        pltpu.VMEM((C, D), jnp.float32),
        pltpu.SemaphoreType.DMA,
        pltpu.SemaphoreType.DMA,
        pltpu.SemaphoreType.DMA,
        pltpu.SemaphoreType.DMA,
    ],
)
def _gather_k(table_hbm, idx_hbm, out_hbm,
              idx0, idx1, rows0, rows1, sg0, sg1, ss0, ss1):
    wid = lax.axis_index("s") * NC + lax.axis_index("c")
    n_mine = (NCHUNK - wid + NW - 1) // NW
    bufs = ((idx0, rows0, sg0, ss0), (idx1, rows1, sg1, ss1))

    def fire_gathers(t, idxb, rowsb, semg):
        j = wid + t * NW
        pltpu.sync_copy(idx_hbm.at[pl.ds(j * KSUB, KSUB), :], idxb)
        return [
            pltpu.async_copy(
                table_hbm.at[idxb.at[k]],
                rowsb.at[pl.ds(k * SUB, SUB), :],
                semg,
            )
            for k in range(KSUB)
        ]

    for b in (0, 1):
        idxb, rowsb, semg, _ = bufs[b]
        fire_gathers(b, idxb, rowsb, semg)

    def body(g, carry):
        for b in (0, 1):
            t = 2 * g + b
            idxb, rowsb, semg, sems = bufs[b]

            @pl.when(t < n_mine)
            def _process():
                for k in range(KSUB):
                    pltpu.make_async_copy(
                        table_hbm.at[idxb.at[k]],
                        rowsb.at[pl.ds(k * SUB, SUB), :],
                        semg,
                    ).wait()
                j = wid + t * NW
                st = pltpu.async_copy(rowsb, out_hbm.at[pl.ds(j * C, C), :], sems)
                st.wait()

                @pl.when(t + 2 < n_mine)
                def _refill():
                    fire_gathers(t + 2, idxb, rowsb, semg)

        return carry

    lax.fori_loop(0, (n_mine + 1) // 2, body, 0)


def kernel(node_features, memory, source_nodes, timestamps, time_w, time_b):
    del timestamps, time_w, time_b
    fused = _fuse_tables(node_features, memory)
    idx = source_nodes.astype(jnp.int32).reshape(NCHUNK * KSUB, SUB)
    return _gather_k(fused, idx)
